# baseline (speedup 1.0000x reference)
import jax
import jax.numpy as jnp
from jax.experimental import pallas as pl
from jax.experimental.pallas import tpu as pltpu

D_MODEL = 768
N_EXPERTS = 64
TOP_K = 2
D_FF = 4 * D_MODEL
T_TOKENS = 2048
N_ASSIGN = T_TOKENS * TOP_K

TBLK = 256
BT = 128
NBLK = N_ASSIGN // BT + N_EXPERTS - 1
NP = NBLK * BT
NBLK_PAD = 128


def _shared_gate_body(x_ref, ws1_ref, bs1_ref, ws2_ref, bs2_ref, wg_ref,
                      bgb_ref, base_ref, i1_ref, i2_ref, w1_ref, w2_ref,
                      r1_ref, r2_ref, cnt_ref, carry_ref):
    b = pl.program_id(0)

    @pl.when(b == 0)
    def _init():
        carry_ref[...] = jnp.zeros_like(carry_ref)

    x = x_ref[...]
    h = x @ ws1_ref[...] + bs1_ref[...]
    h = h * jax.nn.sigmoid(h)
    base_ref[...] = x + h @ ws2_ref[...] + bs2_ref[...]

    logits = x @ wg_ref[...] + bgb_ref[...]
    s = jax.nn.sigmoid(logits)
    lane = jax.lax.broadcasted_iota(jnp.int32, s.shape, 1)
    big = jnp.int32(N_EXPERTS)
    m1 = jnp.max(s, axis=1, keepdims=True)
    i1 = jnp.min(jnp.where(s == m1, lane, big), axis=1, keepdims=True)
    s2 = jnp.where(lane == i1, -jnp.inf, s)
    m2 = jnp.max(s2, axis=1, keepdims=True)
    i2 = jnp.min(jnp.where(s2 == m2, lane, big), axis=1, keepdims=True)
    i1_ref[...] = i1
    i2_ref[...] = i2
    w1_ref[...] = jax.nn.sigmoid(m1 - m2)
    w2_ref[...] = jax.nn.sigmoid(m2 - m1)

    oh1 = (lane == i1).astype(jnp.float32)
    oh2 = (lane == i2).astype(jnp.float32)
    ohsum = oh1 + oh2
    r_io = jax.lax.broadcasted_iota(jnp.int32, (TBLK, TBLK), 0)
    c_io = jax.lax.broadcasted_iota(jnp.int32, (TBLK, TBLK), 1)
    ltri = (r_io > c_io).astype(jnp.float32)
    bx = jax.lax.dot(ltri, ohsum)
    carry = carry_ref[...]
    tot_excl = bx + carry
    r1_ref[...] = jnp.sum(oh1 * tot_excl, axis=1, keepdims=True)
    r2_ref[...] = jnp.sum(oh2 * (tot_excl + oh1), axis=1, keepdims=True)
    new_carry = carry + jnp.sum(ohsum, axis=0, keepdims=True)
    carry_ref[...] = new_carry
    cnt_ref[...] = new_carry


def _finalize_body(cnt_ref, i1_ref, i2_ref, r1_ref, r2_ref,
                   pos1_ref, pos2_ref, blke_ref):
    cnt = cnt_ref[...]
    nb_e = jnp.floor((cnt + (BT - 1)) * (1.0 / BT))
    e_r = jax.lax.broadcasted_iota(jnp.int32, (N_EXPERTS, N_EXPERTS), 0)
    e_c = jax.lax.broadcasted_iota(jnp.int32, (N_EXPERTS, N_EXPERTS), 1)
    utri = (e_r <= e_c).astype(jnp.float32)
    nb_csum = jax.lax.dot(nb_e, utri)
    pstart = (nb_csum - nb_e) * float(BT)

    lane1 = jax.lax.broadcasted_iota(jnp.int32, (T_TOKENS, N_EXPERTS), 1)
    oh1 = (lane1 == i1_ref[...]).astype(jnp.float32)
    oh2 = (lane1 == i2_ref[...]).astype(jnp.float32)
    pos1 = jnp.sum(oh1 * pstart, axis=1, keepdims=True) + r1_ref[...]
    pos2 = jnp.sum(oh2 * pstart, axis=1, keepdims=True) + r2_ref[...]
    pos1_ref[...] = pos1.astype(jnp.int32)
    pos2_ref[...] = pos2.astype(jnp.int32)

    j_io = jax.lax.broadcasted_iota(
        jnp.int32, (NBLK_PAD, N_EXPERTS), 0).astype(jnp.float32)
    ge = (j_io >= nb_csum).astype(jnp.float32)
    blke = jnp.minimum(jnp.sum(ge, axis=1, keepdims=True),
                       float(N_EXPERTS - 1))
    blke_ref[...] = blke.astype(jnp.int32)


def _expert_ffn_body(blk_e_ref, xs_ref, we1_ref, be1_ref, we2_ref, be2_ref,
                     rw_ref, y_ref):
    xg = xs_ref[...]
    h = xg @ we1_ref[0] + be1_ref[0]
    h = h * jax.nn.sigmoid(h)
    y_ref[...] = (h @ we2_ref[0] + be2_ref[0]) * rw_ref[...]


def kernel(x, Ws1, bs1, Ws2, bs2, We1, be1, We2, be2, Wg, bg, bias):
    B, T, d = x.shape
    xf = x.reshape(T, d)

    grid_a = (T // TBLK,)
    base, i1, i2, w1, w2, r1, r2, cnt = pl.pallas_call(
        _shared_gate_body,
        grid=grid_a,
        in_specs=[
            pl.BlockSpec((TBLK, d), lambda b: (b, 0)),
            pl.BlockSpec((d, D_FF), lambda b: (0, 0)),
            pl.BlockSpec((1, D_FF), lambda b: (0, 0)),
            pl.BlockSpec((D_FF, d), lambda b: (0, 0)),
            pl.BlockSpec((1, d), lambda b: (0, 0)),
            pl.BlockSpec((d, N_EXPERTS), lambda b: (0, 0)),
            pl.BlockSpec((1, N_EXPERTS), lambda b: (0, 0)),
        ],
        out_specs=[
            pl.BlockSpec((TBLK, d), lambda b: (b, 0)),
            pl.BlockSpec((TBLK, 1), lambda b: (b, 0)),
            pl.BlockSpec((TBLK, 1), lambda b: (b, 0)),
            pl.BlockSpec((TBLK, 1), lambda b: (b, 0)),
            pl.BlockSpec((TBLK, 1), lambda b: (b, 0)),
            pl.BlockSpec((TBLK, 1), lambda b: (b, 0)),
            pl.BlockSpec((TBLK, 1), lambda b: (b, 0)),
            pl.BlockSpec((1, N_EXPERTS), lambda b: (0, 0)),
        ],
        out_shape=[
            jax.ShapeDtypeStruct((T, d), jnp.float32),
            jax.ShapeDtypeStruct((T, 1), jnp.int32),
            jax.ShapeDtypeStruct((T, 1), jnp.int32),
            jax.ShapeDtypeStruct((T, 1), jnp.float32),
            jax.ShapeDtypeStruct((T, 1), jnp.float32),
            jax.ShapeDtypeStruct((T, 1), jnp.float32),
            jax.ShapeDtypeStruct((T, 1), jnp.float32),
            jax.ShapeDtypeStruct((1, N_EXPERTS), jnp.float32),
        ],
        scratch_shapes=[pltpu.VMEM((1, N_EXPERTS), jnp.float32)],
    )(xf, Ws1, bs1.reshape(1, D_FF), Ws2, bs2.reshape(1, d), Wg,
      (bg + bias).reshape(1, N_EXPERTS))

    pos1, pos2, blke = pl.pallas_call(
        _finalize_body,
        out_shape=[
            jax.ShapeDtypeStruct((T, 1), jnp.int32),
            jax.ShapeDtypeStruct((T, 1), jnp.int32),
            jax.ShapeDtypeStruct((NBLK_PAD, 1), jnp.int32),
        ],
    )(cnt, i1, i2, r1, r2)
    p0 = pos1[:, 0]
    p1 = pos2[:, 0]
    blk_e = blke[:NBLK, 0]

    tokv = jnp.arange(T, dtype=jnp.int32)
    row_tok = (jnp.zeros((NP,), jnp.int32).at[p0].set(tokv)
               .at[p1].set(tokv))
    row_w = (jnp.zeros((NP,), jnp.float32).at[p0].set(w1[:, 0])
             .at[p1].set(w2[:, 0]))
    xs = xf[row_tok]

    grid_spec = pltpu.PrefetchScalarGridSpec(
        num_scalar_prefetch=1,
        grid=(NBLK,),
        in_specs=[
            pl.BlockSpec((BT, d), lambda b, s: (b, 0)),
            pl.BlockSpec((1, d, D_FF), lambda b, s: (s[b], 0, 0)),
            pl.BlockSpec((1, 1, D_FF), lambda b, s: (s[b], 0, 0)),
            pl.BlockSpec((1, D_FF, d), lambda b, s: (s[b], 0, 0)),
            pl.BlockSpec((1, 1, d), lambda b, s: (s[b], 0, 0)),
            pl.BlockSpec((BT, 1), lambda b, s: (b, 0)),
        ],
        out_specs=pl.BlockSpec((BT, d), lambda b, s: (b, 0)),
    )
    y = pl.pallas_call(
        _expert_ffn_body,
        grid_spec=grid_spec,
        out_shape=jax.ShapeDtypeStruct((NP, d), jnp.float32),
        compiler_params=pltpu.CompilerParams(
            vmem_limit_bytes=100 * 1024 * 1024),
    )(blk_e, xs, We1, be1.reshape(N_EXPERTS, 1, D_FF), We2,
      be2.reshape(N_EXPERTS, 1, d), row_w.reshape(NP, 1))

    out = base + y[p0] + y[p1]
    return out.reshape(B, T, d)


# speedup vs baseline: 1.1167x; 1.1167x over previous
import functools

import jax
import jax.numpy as jnp
from jax import lax
from jax.experimental import pallas as pl
from jax.experimental.pallas import tpu as pltpu
from jax.experimental.pallas import tpu_sc as plsc

D_MODEL = 768
N_EXPERTS = 64
TOP_K = 2
D_FF = 4 * D_MODEL
T_TOKENS = 2048
N_ASSIGN = T_TOKENS * TOP_K

TBLK = 256
BT = 128
NBLK = N_ASSIGN // BT + N_EXPERTS - 1
NP = NBLK * BT
NBLK_PAD = 128


def _shared_gate_body(x_ref, ws1_ref, bs1_ref, ws2_ref, bs2_ref, wg_ref,
                      bgb_ref, base_ref, i1_ref, i2_ref, w1_ref, w2_ref,
                      r1_ref, r2_ref, cnt_ref, carry_ref):
    b = pl.program_id(0)

    @pl.when(b == 0)
    def _init():
        carry_ref[...] = jnp.zeros_like(carry_ref)

    x = x_ref[...]
    h = x @ ws1_ref[...] + bs1_ref[...]
    h = h * jax.nn.sigmoid(h)
    base_ref[...] = x + h @ ws2_ref[...] + bs2_ref[...]

    logits = x @ wg_ref[...] + bgb_ref[...]
    s = jax.nn.sigmoid(logits)
    lane = jax.lax.broadcasted_iota(jnp.int32, s.shape, 1)
    big = jnp.int32(N_EXPERTS)
    m1 = jnp.max(s, axis=1, keepdims=True)
    i1 = jnp.min(jnp.where(s == m1, lane, big), axis=1, keepdims=True)
    s2 = jnp.where(lane == i1, -jnp.inf, s)
    m2 = jnp.max(s2, axis=1, keepdims=True)
    i2 = jnp.min(jnp.where(s2 == m2, lane, big), axis=1, keepdims=True)
    i1_ref[...] = i1
    i2_ref[...] = i2
    w1_ref[...] = jax.nn.sigmoid(m1 - m2)
    w2_ref[...] = jax.nn.sigmoid(m2 - m1)

    oh1 = (lane == i1).astype(jnp.float32)
    oh2 = (lane == i2).astype(jnp.float32)
    ohsum = oh1 + oh2
    r_io = jax.lax.broadcasted_iota(jnp.int32, (TBLK, TBLK), 0)
    c_io = jax.lax.broadcasted_iota(jnp.int32, (TBLK, TBLK), 1)
    ltri = (r_io > c_io).astype(jnp.float32)
    bx = jax.lax.dot(ltri, ohsum)
    carry = carry_ref[...]
    tot_excl = bx + carry
    r1_ref[...] = jnp.sum(oh1 * tot_excl, axis=1, keepdims=True)
    r2_ref[...] = jnp.sum(oh2 * (tot_excl + oh1), axis=1, keepdims=True)
    new_carry = carry + jnp.sum(ohsum, axis=0, keepdims=True)
    carry_ref[...] = new_carry
    cnt_ref[...] = new_carry


def _finalize_body(cnt_ref, i1_ref, i2_ref, r1_ref, r2_ref, w1_ref, w2_ref,
                   pos1_ref, pos2_ref, blke_ref, w1b_ref, w2b_ref):
    ones16 = jnp.ones((1, 16), jnp.float32)
    w1b_ref[...] = w1_ref[...] * ones16
    w2b_ref[...] = w2_ref[...] * ones16
    cnt = cnt_ref[...]
    nb_e = jnp.floor((cnt + (BT - 1)) * (1.0 / BT))
    e_r = jax.lax.broadcasted_iota(jnp.int32, (N_EXPERTS, N_EXPERTS), 0)
    e_c = jax.lax.broadcasted_iota(jnp.int32, (N_EXPERTS, N_EXPERTS), 1)
    utri = (e_r <= e_c).astype(jnp.float32)
    nb_csum = jax.lax.dot(nb_e, utri)
    pstart = (nb_csum - nb_e) * float(BT)

    lane1 = jax.lax.broadcasted_iota(jnp.int32, (T_TOKENS, N_EXPERTS), 1)
    oh1 = (lane1 == i1_ref[...]).astype(jnp.float32)
    oh2 = (lane1 == i2_ref[...]).astype(jnp.float32)
    pos1 = jnp.sum(oh1 * pstart, axis=1, keepdims=True) + r1_ref[...]
    pos2 = jnp.sum(oh2 * pstart, axis=1, keepdims=True) + r2_ref[...]
    pos1_ref[...] = pos1.astype(jnp.int32)
    pos2_ref[...] = pos2.astype(jnp.int32)

    j_io = jax.lax.broadcasted_iota(
        jnp.int32, (NBLK_PAD, N_EXPERTS), 0).astype(jnp.float32)
    ge = (j_io >= nb_csum).astype(jnp.float32)
    blke = jnp.minimum(jnp.sum(ge, axis=1, keepdims=True),
                       float(N_EXPERTS - 1))
    blke_ref[...] = blke.astype(jnp.int32)


def _expert_ffn_body(blk_e_ref, xs_ref, we1_ref, be1_ref, we2_ref, be2_ref,
                     y_ref):
    xg = xs_ref[...]
    h = xg @ we1_ref[0] + be1_ref[0]
    h = h * jax.nn.sigmoid(h)
    y_ref[...] = h @ we2_ref[0] + be2_ref[0]


_SC_INFO = plsc.get_sparse_core_info()
_NWORK = _SC_INFO.num_cores * _SC_INFO.num_subcores
TPW = T_TOKENS // _NWORK
CCH = TPW // 2


def _dispatch_sc(xf_hbm, pos1_hbm, pos2_hbm, xs_hbm, idx1_v, idx2_v, xbuf,
                 sem):
    wid = lax.axis_index("s") * _SC_INFO.num_cores + lax.axis_index("c")
    start = wid * TPW
    pltpu.sync_copy(pos1_hbm.at[pl.ds(start, TPW)], idx1_v)
    pltpu.sync_copy(pos2_hbm.at[pl.ds(start, TPW)], idx2_v)
    pltpu.sync_copy(xf_hbm.at[pl.ds(start, TPW)], xbuf)
    pltpu.async_copy(xbuf, xs_hbm.at[idx1_v], sem).wait()
    pltpu.async_copy(xbuf, xs_hbm.at[idx2_v], sem).wait()


def _combine_sc(base_hbm, y_hbm, pos1_hbm, pos2_hbm, w1b_hbm, w2b_hbm,
                out_hbm, idx1_v, idx2_v, w1_v, w2_v, y1_buf, y2_buf, ob_buf,
                sem):
    wid = lax.axis_index("s") * _SC_INFO.num_cores + lax.axis_index("c")
    start = wid * TPW

    def chunk(c, _):
        cstart = start + c * CCH
        pltpu.sync_copy(pos1_hbm.at[pl.ds(cstart, CCH)], idx1_v)
        pltpu.sync_copy(pos2_hbm.at[pl.ds(cstart, CCH)], idx2_v)
        pltpu.sync_copy(w1b_hbm.at[pl.ds(cstart, CCH)], w1_v)
        pltpu.sync_copy(w2b_hbm.at[pl.ds(cstart, CCH)], w2_v)
        pltpu.async_copy(y_hbm.at[idx1_v], y1_buf, sem).wait()
        pltpu.async_copy(y_hbm.at[idx2_v], y2_buf, sem).wait()
        pltpu.sync_copy(base_hbm.at[pl.ds(cstart, CCH)], ob_buf)

        def per_token(i, _):
            w1s = w1_v[i, pl.ds(0, 16)]
            w2s = w2_v[i, pl.ds(0, 16)]

            def per_vec(j, _):
                sl = (i, pl.ds(j * 16, 16))
                ob_buf[sl] = (ob_buf[sl] + w1s * y1_buf[sl]
                              + w2s * y2_buf[sl])
                return 0

            return lax.fori_loop(0, D_MODEL // 16, per_vec, 0, unroll=8)

        lax.fori_loop(0, CCH, per_token, 0)
        pltpu.sync_copy(ob_buf, out_hbm.at[pl.ds(cstart, CCH)])
        return 0

    lax.fori_loop(0, TPW // CCH, chunk, 0)


def _run_dispatch(xf, pos1, pos2):
    mesh = plsc.VectorSubcoreMesh(core_axis_name="c", subcore_axis_name="s")
    k = functools.partial(
        pl.kernel,
        out_type=jax.ShapeDtypeStruct((NP, D_MODEL), jnp.float32),
        mesh=mesh,
        scratch_types=[
            pltpu.VMEM((TPW,), jnp.int32),
            pltpu.VMEM((TPW,), jnp.int32),
            pltpu.VMEM((TPW, D_MODEL), jnp.float32),
            pltpu.SemaphoreType.DMA,
        ],
    )(_dispatch_sc)
    return k(xf, pos1, pos2)


def _run_combine(base, y, pos1, pos2, w1b, w2b):
    mesh = plsc.VectorSubcoreMesh(core_axis_name="c", subcore_axis_name="s")
    k = functools.partial(
        pl.kernel,
        out_type=jax.ShapeDtypeStruct((T_TOKENS, D_MODEL), jnp.float32),
        mesh=mesh,
        scratch_types=[
            pltpu.VMEM((CCH,), jnp.int32),
            pltpu.VMEM((CCH,), jnp.int32),
            pltpu.VMEM((CCH, 16), jnp.float32),
            pltpu.VMEM((CCH, 16), jnp.float32),
            pltpu.VMEM((CCH, D_MODEL), jnp.float32),
            pltpu.VMEM((CCH, D_MODEL), jnp.float32),
            pltpu.VMEM((CCH, D_MODEL), jnp.float32),
            pltpu.SemaphoreType.DMA,
        ],
    )(_combine_sc)
    return k(base, y, pos1, pos2, w1b, w2b)


def kernel(x, Ws1, bs1, Ws2, bs2, We1, be1, We2, be2, Wg, bg, bias):
    B, T, d = x.shape
    xf = x.reshape(T, d)

    grid_a = (T // TBLK,)
    base, i1, i2, w1, w2, r1, r2, cnt = pl.pallas_call(
        _shared_gate_body,
        grid=grid_a,
        in_specs=[
            pl.BlockSpec((TBLK, d), lambda b: (b, 0)),
            pl.BlockSpec((d, D_FF), lambda b: (0, 0)),
            pl.BlockSpec((1, D_FF), lambda b: (0, 0)),
            pl.BlockSpec((D_FF, d), lambda b: (0, 0)),
            pl.BlockSpec((1, d), lambda b: (0, 0)),
            pl.BlockSpec((d, N_EXPERTS), lambda b: (0, 0)),
            pl.BlockSpec((1, N_EXPERTS), lambda b: (0, 0)),
        ],
        out_specs=[
            pl.BlockSpec((TBLK, d), lambda b: (b, 0)),
            pl.BlockSpec((TBLK, 1), lambda b: (b, 0)),
            pl.BlockSpec((TBLK, 1), lambda b: (b, 0)),
            pl.BlockSpec((TBLK, 1), lambda b: (b, 0)),
            pl.BlockSpec((TBLK, 1), lambda b: (b, 0)),
            pl.BlockSpec((TBLK, 1), lambda b: (b, 0)),
            pl.BlockSpec((TBLK, 1), lambda b: (b, 0)),
            pl.BlockSpec((1, N_EXPERTS), lambda b: (0, 0)),
        ],
        out_shape=[
            jax.ShapeDtypeStruct((T, d), jnp.float32),
            jax.ShapeDtypeStruct((T, 1), jnp.int32),
            jax.ShapeDtypeStruct((T, 1), jnp.int32),
            jax.ShapeDtypeStruct((T, 1), jnp.float32),
            jax.ShapeDtypeStruct((T, 1), jnp.float32),
            jax.ShapeDtypeStruct((T, 1), jnp.float32),
            jax.ShapeDtypeStruct((T, 1), jnp.float32),
            jax.ShapeDtypeStruct((1, N_EXPERTS), jnp.float32),
        ],
        scratch_shapes=[pltpu.VMEM((1, N_EXPERTS), jnp.float32)],
    )(xf, Ws1, bs1.reshape(1, D_FF), Ws2, bs2.reshape(1, d), Wg,
      (bg + bias).reshape(1, N_EXPERTS))

    pos1, pos2, blke, w1b, w2b = pl.pallas_call(
        _finalize_body,
        out_shape=[
            jax.ShapeDtypeStruct((T, 1), jnp.int32),
            jax.ShapeDtypeStruct((T, 1), jnp.int32),
            jax.ShapeDtypeStruct((NBLK_PAD, 1), jnp.int32),
            jax.ShapeDtypeStruct((T, 16), jnp.float32),
            jax.ShapeDtypeStruct((T, 16), jnp.float32),
        ],
    )(cnt, i1, i2, r1, r2, w1, w2)
    p0 = pos1[:, 0]
    p1 = pos2[:, 0]
    blk_e = blke[:NBLK, 0]

    xs = _run_dispatch(xf, p0, p1)

    grid_spec = pltpu.PrefetchScalarGridSpec(
        num_scalar_prefetch=1,
        grid=(NBLK,),
        in_specs=[
            pl.BlockSpec((BT, d), lambda b, s: (b, 0)),
            pl.BlockSpec((1, d, D_FF), lambda b, s: (s[b], 0, 0)),
            pl.BlockSpec((1, 1, D_FF), lambda b, s: (s[b], 0, 0)),
            pl.BlockSpec((1, D_FF, d), lambda b, s: (s[b], 0, 0)),
            pl.BlockSpec((1, 1, d), lambda b, s: (s[b], 0, 0)),
        ],
        out_specs=pl.BlockSpec((BT, d), lambda b, s: (b, 0)),
    )
    y = pl.pallas_call(
        _expert_ffn_body,
        grid_spec=grid_spec,
        out_shape=jax.ShapeDtypeStruct((NP, d), jnp.float32),
        compiler_params=pltpu.CompilerParams(
            vmem_limit_bytes=100 * 1024 * 1024),
    )(blk_e, xs, We1, be1.reshape(N_EXPERTS, 1, D_FF), We2,
      be2.reshape(N_EXPERTS, 1, d))

    out = _run_combine(base, y, p0, p1, w1b, w2b)
    return out.reshape(B, T, d)
